# baseline (speedup 1.0000x reference)
import functools

import jax
import jax.numpy as jnp
from jax import lax
from jax.experimental import pallas as pl
from jax.experimental.pallas import tpu as pltpu
from jax.experimental.pallas import tpu_sc as plsc

N = 100000
K = 16
D = 128
T = 512
RW = D // 2
NC = 2
NS = 16
NW = NC * NS
TB = 160
NTILES = N // TB
G = D // 32
H1 = 80


def _body(x_hbm, cw_hbm, tab_hbm, out_hbm,
          tab_v, x_v, cw_v, in0, in1, out0, out1):
    wid = lax.axis_index("s") * NC + lax.axis_index("c")
    cnt = jnp.where(wid < NTILES % NW, NTILES // NW + 1, NTILES // NW)
    pltpu.sync_copy(tab_hbm, tab_v)
    in_sems = (in0, in1)
    out_sems = (out0, out1)

    def in_descs(j, b):
        row0 = (wid + j * NW) * TB
        return (
            pltpu.make_async_copy(x_hbm.at[pl.ds(row0, TB)],
                                  x_v.at[pl.ds(b * TB, TB)], in_sems[b]),
            pltpu.make_async_copy(cw_hbm.at[pl.ds(row0, TB)],
                                  cw_v.at[pl.ds(b * TB, TB)], in_sems[b]),
        )

    def out_desc(j, b):
        return pltpu.make_async_copy(
            x_v.at[pl.ds(b * TB, TB)],
            out_hbm.at[pl.ds((wid + j * NW) * TB, TB)],
            out_sems[b])

    def compute(b, lo, hi):
        @plsc.parallel_loop(lo, hi, unroll=2)
        def node_body(n):
            nn = n + b * TB
            cvec = cw_v[nn, :]
            iv = (cvec & 0x1FF) << 6
            wclean = (cvec & -65536) | lax.shift_right_logical(cvec, 16)
            bases = [iv[k] for k in range(K)]
            wks = [plsc.bitcast(jnp.broadcast_to(wclean[k], (16,)),
                                jnp.bfloat16) for k in range(K)]

            def trow(k, g):
                return plsc.bitcast(tab_v[pl.ds(bases[k] + g * 16, 16)],
                                    jnp.bfloat16)

            for g in range(G):
                acc0 = wks[0] * trow(0, g)
                acc1 = wks[1] * trow(1, g)
                for k in range(2, K, 2):
                    acc0 = acc0 + wks[k] * trow(k, g)
                    acc1 = acc1 + wks[k + 1] * trow(k + 1, g)
                a, bb = plsc.unpack(acc0 + acc1,
                                    format=plsc.PackFormat.INTERLEAVED)
                sa = pl.ds(g * 32, 16)
                sb = pl.ds(g * 32 + 16, 16)
                x_v[nn, sa] = x_v[nn, sa] + a
                x_v[nn, sb] = x_v[nn, sb] + bb

    def process(j, b):
        nb = 1 - b
        for dsc in in_descs(j, b):
            dsc.wait()
        compute(b, 0, H1)
        @pl.when(j >= 1)
        def _():
            out_desc(j - 1, nb).wait()

        @pl.when(j + 1 < cnt)
        def _():
            for dsc in in_descs(j + 1, nb):
                dsc.start()

        compute(b, H1, TB)
        out_desc(j, b).start()

    for dsc in in_descs(0, 0):
        dsc.start()

    def pair(p, carry):
        process(2 * p, 0)
        process(2 * p + 1, 1)
        return carry

    lax.fori_loop(0, cnt // 2, pair, 0)

    @pl.when(cnt % 2 == 1)
    def _():
        process(cnt - 1, 0)
        out_desc(cnt - 1, 0).wait()

    @pl.when(cnt % 2 == 0)
    def _():
        out_desc(cnt - 1, 1).wait()


def kernel(x, degree_weights, degree_indices, degree_table):
    wb = jax.lax.bitcast_convert_type(
        degree_weights.astype(jnp.bfloat16), jnp.uint16).astype(jnp.uint32)
    cw = jax.lax.bitcast_convert_type(
        (wb << 16) | degree_indices.astype(jnp.uint32), jnp.int32)
    tabp = (degree_table.reshape(T, G, 2, 16)
            .transpose(0, 1, 3, 2)
            .reshape(T, RW, 2)
            .astype(jnp.bfloat16))
    tabi = jax.lax.bitcast_convert_type(tabp, jnp.int32).reshape(T * RW)
    mesh = plsc.VectorSubcoreMesh(core_axis_name="c", subcore_axis_name="s")
    f = functools.partial(
        pl.kernel,
        out_type=jax.ShapeDtypeStruct((N, D), jnp.float32),
        mesh=mesh,
        compiler_params=pltpu.CompilerParams(
            use_tc_tiling_on_sc=True, needs_layout_passes=False),
        scratch_types=[
            pltpu.VMEM((T * RW,), jnp.int32),
            pltpu.VMEM((2 * TB, D), jnp.float32),
            pltpu.VMEM((2 * TB, K), jnp.int32),
            pltpu.SemaphoreType.DMA,
            pltpu.SemaphoreType.DMA,
            pltpu.SemaphoreType.DMA,
            pltpu.SemaphoreType.DMA,
        ],
    )(_body)
    return f(x, cw, tabi)


# speedup vs baseline: 1.0440x; 1.0440x over previous
import functools

import jax
import jax.numpy as jnp
from jax import lax
from jax.experimental import pallas as pl
from jax.experimental.pallas import tpu as pltpu
from jax.experimental.pallas import tpu_sc as plsc

N = 100000
K = 16
D = 128
T = 512
RW = D // 2
NC = 2
NS = 16
NW = NC * NS
TB = 160
NTILES = N // TB
G = D // 32
H1 = 80


def _body(x_hbm, wi_hbm, tab_hbm, out_hbm,
          tab_v, x_v, w_v, idx_v, in0, in1, out0, out1):
    wid = lax.axis_index("s") * NC + lax.axis_index("c")
    cnt = jnp.where(wid < NTILES % NW, NTILES // NW + 1, NTILES // NW)
    pltpu.sync_copy(tab_hbm, tab_v)
    in_sems = (in0, in1)
    out_sems = (out0, out1)

    def in_descs(j, b):
        row0 = (wid + j * NW) * TB
        return (
            pltpu.make_async_copy(x_hbm.at[pl.ds(row0, TB)],
                                  x_v.at[pl.ds(b * TB, TB)], in_sems[b]),
            pltpu.make_async_copy(wi_hbm.at[pl.ds(row0 * K, TB * K)],
                                  w_v.at[pl.ds(b * TB * K, TB * K)],
                                  in_sems[b]),
            pltpu.make_async_copy(wi_hbm.at[pl.ds(N * K + row0 * K, TB * K)],
                                  idx_v.at[pl.ds(b * TB * K, TB * K)],
                                  in_sems[b]),
        )

    def out_desc(j, b):
        return pltpu.make_async_copy(
            x_v.at[pl.ds(b * TB, TB)],
            out_hbm.at[pl.ds((wid + j * NW) * TB, TB)],
            out_sems[b])

    def compute(b, lo, hi):
        @plsc.parallel_loop(lo, hi, unroll=2)
        def node_body(n):
            nn = n + b * TB
            ivec = idx_v[pl.ds(nn * K, 16)]
            wvec = w_v[pl.ds(nn * K, 16)]
            bases = [ivec[k] * RW for k in range(K)]
            wks = [plsc.bitcast(jnp.broadcast_to(wvec[k], (16,)),
                                jnp.bfloat16) for k in range(K)]

            def trow(k, g):
                return plsc.bitcast(tab_v[pl.ds(bases[k] + g * 16, 16)],
                                    jnp.bfloat16)

            for g in range(G):
                acc0 = wks[0] * trow(0, g)
                acc1 = wks[1] * trow(1, g)
                for k in range(2, K, 2):
                    acc0 = acc0 + wks[k] * trow(k, g)
                    acc1 = acc1 + wks[k + 1] * trow(k + 1, g)
                a, bb = plsc.unpack(acc0 + acc1,
                                    format=plsc.PackFormat.INTERLEAVED)
                sa = pl.ds(g * 32, 16)
                sb = pl.ds(g * 32 + 16, 16)
                x_v[nn, sa] = x_v[nn, sa] + a
                x_v[nn, sb] = x_v[nn, sb] + bb

    def process(j, b):
        nb = 1 - b
        for dsc in in_descs(j, b):
            dsc.wait()
        compute(b, 0, H1)
        @pl.when(j >= 1)
        def _():
            out_desc(j - 1, nb).wait()

        @pl.when(j + 1 < cnt)
        def _():
            for dsc in in_descs(j + 1, nb):
                dsc.start()

        compute(b, H1, TB)
        out_desc(j, b).start()

    for dsc in in_descs(0, 0):
        dsc.start()

    def pair(p, carry):
        process(2 * p, 0)
        process(2 * p + 1, 1)
        return carry

    lax.fori_loop(0, cnt // 2, pair, 0)

    @pl.when(cnt % 2 == 1)
    def _():
        process(cnt - 1, 0)
        out_desc(cnt - 1, 0).wait()

    @pl.when(cnt % 2 == 0)
    def _():
        out_desc(cnt - 1, 1).wait()


def kernel(x, degree_weights, degree_indices, degree_table):
    wb = jax.lax.bitcast_convert_type(
        degree_weights.astype(jnp.bfloat16), jnp.uint16).astype(jnp.uint32)
    wp = jax.lax.bitcast_convert_type(wb | (wb << 16), jnp.int32).reshape(N * K)
    idx = degree_indices.astype(jnp.int32).reshape(N * K)
    wi = jnp.concatenate([wp, idx])
    tabp = (degree_table.reshape(T, G, 2, 16)
            .transpose(0, 1, 3, 2)
            .reshape(T, RW, 2)
            .astype(jnp.bfloat16))
    tabi = jax.lax.bitcast_convert_type(tabp, jnp.int32).reshape(T * RW)
    mesh = plsc.VectorSubcoreMesh(core_axis_name="c", subcore_axis_name="s")
    f = functools.partial(
        pl.kernel,
        out_type=jax.ShapeDtypeStruct((N, D), jnp.float32),
        mesh=mesh,
        compiler_params=pltpu.CompilerParams(
            use_tc_tiling_on_sc=True, needs_layout_passes=False),
        scratch_types=[
            pltpu.VMEM((T * RW,), jnp.int32),
            pltpu.VMEM((2 * TB, D), jnp.float32),
            pltpu.VMEM((2 * TB * K,), jnp.int32),
            pltpu.VMEM((2 * TB * K,), jnp.int32),
            pltpu.SemaphoreType.DMA,
            pltpu.SemaphoreType.DMA,
            pltpu.SemaphoreType.DMA,
            pltpu.SemaphoreType.DMA,
        ],
    )(_body)
    return f(x, wi, tabi)


# speedup vs baseline: 1.1369x; 1.0890x over previous
import functools

import jax
import jax.numpy as jnp
from jax import lax
from jax.experimental import pallas as pl
from jax.experimental.pallas import tpu as pltpu
from jax.experimental.pallas import tpu_sc as plsc

N = 100000
K = 16
D = 128
T = 512
RW = D // 2
NC = 2
NS = 16
NW = NC * NS
TB = 160
NTILES = N // TB
G = D // 32
H1 = 80


def _body(x_hbm, w_hbm, tab_hbm, out_hbm,
          tab_v, x_v, w_v, in0, in1, out0, out1):
    wid = lax.axis_index("s") * NC + lax.axis_index("c")
    cnt = jnp.where(wid < NTILES % NW, NTILES // NW + 1, NTILES // NW)
    pltpu.sync_copy(tab_hbm, tab_v)
    in_sems = (in0, in1)
    out_sems = (out0, out1)

    def in_descs(j, b):
        row0 = (wid + j * NW) * TB
        return (
            pltpu.make_async_copy(x_hbm.at[pl.ds(row0, TB)],
                                  x_v.at[pl.ds(b * TB, TB)], in_sems[b]),
            pltpu.make_async_copy(w_hbm.at[pl.ds(row0 * K, TB * K)],
                                  w_v.at[pl.ds(b * TB * K, TB * K)],
                                  in_sems[b]),
        )

    def out_desc(j, b):
        return pltpu.make_async_copy(
            x_v.at[pl.ds(b * TB, TB)],
            out_hbm.at[pl.ds((wid + j * NW) * TB, TB)],
            out_sems[b])

    def compute(b, lo, hi):
        @plsc.parallel_loop(lo, hi, unroll=2)
        def node_body(n):
            nn = n + b * TB
            cvec = w_v[pl.ds(nn * K, 16)]
            wclean = ((cvec & -65536)
                      | lax.shift_right_logical(cvec, 16))
            bases = [(cvec[k] & 0x1FF) * RW for k in range(K)]
            wks = [plsc.bitcast(jnp.broadcast_to(wclean[k], (16,)),
                                jnp.bfloat16) for k in range(K)]

            def trow(k, g):
                return plsc.bitcast(tab_v[pl.ds(bases[k] + g * 16, 16)],
                                    jnp.bfloat16)

            for g in range(G):
                acc0 = wks[0] * trow(0, g)
                acc1 = wks[1] * trow(1, g)
                for k in range(2, K, 2):
                    acc0 = acc0 + wks[k] * trow(k, g)
                    acc1 = acc1 + wks[k + 1] * trow(k + 1, g)
                a, bb = plsc.unpack(acc0 + acc1,
                                    format=plsc.PackFormat.INTERLEAVED)
                sa = pl.ds(g * 32, 16)
                sb = pl.ds(g * 32 + 16, 16)
                x_v[nn, sa] = x_v[nn, sa] + a
                x_v[nn, sb] = x_v[nn, sb] + bb

    def process(j, b):
        nb = 1 - b
        for dsc in in_descs(j, b):
            dsc.wait()
        compute(b, 0, H1)
        @pl.when(j >= 1)
        def _():
            out_desc(j - 1, nb).wait()

        @pl.when(j + 1 < cnt)
        def _():
            for dsc in in_descs(j + 1, nb):
                dsc.start()

        compute(b, H1, TB)
        out_desc(j, b).start()

    for dsc in in_descs(0, 0):
        dsc.start()

    def pair(p, carry):
        process(2 * p, 0)
        process(2 * p + 1, 1)
        return carry

    lax.fori_loop(0, cnt // 2, pair, 0)

    @pl.when(cnt % 2 == 1)
    def _():
        process(cnt - 1, 0)
        out_desc(cnt - 1, 0).wait()

    @pl.when(cnt % 2 == 0)
    def _():
        out_desc(cnt - 1, 1).wait()


def kernel(x, degree_weights, degree_indices, degree_table):
    wb = jax.lax.bitcast_convert_type(
        degree_weights.astype(jnp.bfloat16), jnp.uint16).astype(jnp.uint32)
    wp = jax.lax.bitcast_convert_type(
        (wb << 16) | degree_indices.astype(jnp.uint32),
        jnp.int32).reshape(N * K)
    tabp = (degree_table.reshape(T, G, 2, 16)
            .transpose(0, 1, 3, 2)
            .reshape(T, RW, 2)
            .astype(jnp.bfloat16))
    tabi = jax.lax.bitcast_convert_type(tabp, jnp.int32).reshape(T * RW)
    mesh = plsc.VectorSubcoreMesh(core_axis_name="c", subcore_axis_name="s")
    f = functools.partial(
        pl.kernel,
        out_type=jax.ShapeDtypeStruct((N, D), jnp.float32),
        mesh=mesh,
        compiler_params=pltpu.CompilerParams(
            use_tc_tiling_on_sc=True, needs_layout_passes=False),
        scratch_types=[
            pltpu.VMEM((T * RW,), jnp.int32),
            pltpu.VMEM((2 * TB, D), jnp.float32),
            pltpu.VMEM((2 * TB * K,), jnp.int32),
            pltpu.SemaphoreType.DMA,
            pltpu.SemaphoreType.DMA,
            pltpu.SemaphoreType.DMA,
            pltpu.SemaphoreType.DMA,
        ],
    )(_body)
    return f(x, wp, tabi)
